# baseline (speedup 1.0000x reference)
import functools
import math

import jax
import jax.numpy as jnp
from jax import lax
from jax.experimental import pallas as pl
from jax.experimental.pallas import tpu as pltpu
from jax.experimental.pallas import tpu_sc as plsc

D_MODEL = 64
SCALE = math.sqrt(D_MODEL)
NUM_CORES = 2
NUM_SUBCORES = 16
NUM_WORKERS = NUM_CORES * NUM_SUBCORES
LANES = 16
CHUNK = 512


def kernel(input_ids, embedding_weight):
    batch, seq = input_ids.shape
    total = batch * seq
    assert total % (NUM_WORKERS * CHUNK) == 0
    per_worker = total // NUM_WORKERS
    n_chunks = per_worker // CHUNK

    idx_flat = input_ids.reshape(total)

    mesh = plsc.VectorSubcoreMesh(
        core_axis_name="c", subcore_axis_name="s",
        num_cores=NUM_CORES, num_subcores=NUM_SUBCORES)

    @functools.partial(
        pl.kernel,
        mesh=mesh,
        out_type=jax.ShapeDtypeStruct((total, D_MODEL), jnp.float32),
        scratch_types=[
            pltpu.VMEM((CHUNK,), jnp.int32),
            pltpu.VMEM((CHUNK, D_MODEL), jnp.float32),
            pltpu.SemaphoreType.DMA,
        ],
        compiler_params=pltpu.CompilerParams(use_tc_tiling_on_sc=False),
    )
    def emb(idx_hbm, table_hbm, out_hbm, idx_v, rows_v, sem):
        wid = lax.axis_index("s") * NUM_CORES + lax.axis_index("c")
        base = wid * per_worker

        def chunk_body(g, carry):
            off = base + g * CHUNK
            pltpu.sync_copy(idx_hbm.at[pl.ds(off, CHUNK)], idx_v)
            pltpu.async_copy(table_hbm.at[idx_v], rows_v, sem).wait()

            def row_body(i, c):
                for j in range(D_MODEL // LANES):
                    sl = pl.ds(j * LANES, LANES)
                    rows_v[i, sl] = rows_v[i, sl] * SCALE
                return c

            lax.fori_loop(0, CHUNK, row_body, 0, unroll=2)
            pltpu.sync_copy(rows_v, out_hbm.at[pl.ds(off, CHUNK)])
            return carry

        lax.fori_loop(0, n_chunks, chunk_body, 0)

    out = emb(idx_flat, embedding_weight)
    return out.reshape(batch, seq, D_MODEL)


# speedup vs baseline: 1.0893x; 1.0893x over previous
import functools
import math

import jax
import jax.numpy as jnp
from jax import lax
from jax.experimental import pallas as pl
from jax.experimental.pallas import tpu as pltpu
from jax.experimental.pallas import tpu_sc as plsc

D_MODEL = 64
SCALE = math.sqrt(D_MODEL)
NUM_CORES = 2
NUM_SUBCORES = 16
NUM_WORKERS = NUM_CORES * NUM_SUBCORES
LANES = 16
CHUNK = 256
NBUF = 4


def kernel(input_ids, embedding_weight):
    batch, seq = input_ids.shape
    total = batch * seq
    assert total % (NUM_WORKERS * CHUNK) == 0
    per_worker = total // NUM_WORKERS
    n_chunks = per_worker // CHUNK
    assert n_chunks % NBUF == 0 and n_chunks >= 2 * NBUF

    idx_flat = input_ids.reshape(total)

    mesh = plsc.VectorSubcoreMesh(
        core_axis_name="c", subcore_axis_name="s",
        num_cores=NUM_CORES, num_subcores=NUM_SUBCORES)

    @functools.partial(
        pl.kernel,
        mesh=mesh,
        out_type=jax.ShapeDtypeStruct((total, D_MODEL), jnp.float32),
        scratch_types=[
            pltpu.VMEM((per_worker,), jnp.int32),
            pltpu.VMEM((NBUF, CHUNK, D_MODEL), jnp.float32),
            [pltpu.SemaphoreType.DMA] * NBUF,
            [pltpu.SemaphoreType.DMA] * NBUF,
        ],
        compiler_params=pltpu.CompilerParams(use_tc_tiling_on_sc=False),
    )
    def emb(idx_hbm, table_hbm, out_hbm, idx_v, rows_v, sem_g, sem_w):
        wid = lax.axis_index("s") * NUM_CORES + lax.axis_index("c")
        base = wid * per_worker

        pltpu.sync_copy(idx_hbm.at[pl.ds(base, per_worker)], idx_v)

        def issue_gather(g, b):
            pltpu.async_copy(
                table_hbm.at[idx_v.at[pl.ds(g * CHUNK, CHUNK)]],
                rows_v.at[b], sem_g[b])

        def wait_gather(g, b):
            pltpu.make_async_copy(
                table_hbm.at[idx_v.at[pl.ds(g * CHUNK, CHUNK)]],
                rows_v.at[b], sem_g[b]).wait()

        def scale_rows(b):
            @plsc.parallel_loop(0, CHUNK, unroll=4)
            def _(i):
                for j in range(D_MODEL // LANES):
                    sl = pl.ds(j * LANES, LANES)
                    rows_v[b, i, sl] = rows_v[b, i, sl] * SCALE

        def issue_write(g, b):
            pltpu.async_copy(
                rows_v.at[b], out_hbm.at[pl.ds(base + g * CHUNK, CHUNK)],
                sem_w[b])

        def wait_write(g, b):
            pltpu.make_async_copy(
                rows_v.at[b], out_hbm.at[pl.ds(base + g * CHUNK, CHUNK)],
                sem_w[b]).wait()

        issue_gather(0, 0)
        issue_gather(1, 1)
        for g in (0, 1):
            wait_gather(g, g)
            scale_rows(g)
            issue_write(g, g)
            issue_gather(g + 2, g + 2)

        def outer(g0, carry):
            for db in range(NBUF):
                b = (2 + db) % NBUF
                gg = g0 + db
                wait_gather(gg, b)
                scale_rows(b)
                issue_write(gg, b)
                wait_write(gg - 2, (b + 2) % NBUF)
                issue_gather(gg + 2, (b + 2) % NBUF)
            return carry

        n_main = (n_chunks - 4) // NBUF
        lax.fori_loop(0, n_main, lambda s, c: outer(2 + s * NBUF, c), 0)

        for gg in (n_chunks - 2, n_chunks - 1):
            b = gg % NBUF
            wait_gather(gg, b)
            scale_rows(b)
            issue_write(gg, b)

        for gg in range(n_chunks - 4, n_chunks):
            wait_write(gg, gg % NBUF)

    out = emb(idx_flat, embedding_weight)
    return out.reshape(batch, seq, D_MODEL)
